# baseline (speedup 1.0000x reference)
import functools

import jax
import jax.numpy as jnp
from jax import lax
from jax.experimental import pallas as pl
from jax.experimental.pallas import tpu as pltpu
from jax.experimental.pallas import tpu_sc as plsc

NUM_EMB = 1_000_000
DIM = 64
BATCH = 16384
HIST = 50

NUM_CORES = 2
NUM_SUBCORES = 16
NW = NUM_CORES * NUM_SUBCORES
CB = 128
NBLK_TOTAL = HIST * (BATCH // CB)
NBLK = NBLK_TOTAL // NW

NIB = 8
NGB = 4

_mesh = plsc.VectorSubcoreMesh(core_axis_name="c", subcore_axis_name="s")


@functools.partial(
    pl.kernel,
    mesh=_mesh,
    out_type=jax.ShapeDtypeStruct((HIST, DIM, BATCH), jnp.float32),
    scratch_types=[
        pltpu.VMEM((NIB, CB), jnp.int32),
        pltpu.VMEM((NGB, CB, 2 * DIM), jnp.float32),
        pltpu.VMEM((NGB, DIM, CB), jnp.float32),
        pltpu.SemaphoreType.DMA((NIB,)),
        pltpu.SemaphoreType.DMA((NGB,)),
        pltpu.SemaphoreType.DMA((NGB,)),
    ],
    compiler_params=pltpu.CompilerParams(needs_layout_passes=False),
)
def _gather_rows(ids_hbm, tab_hbm, out_hbm, idx_v, gb_v, ob_v, isem, gsem, ssem):
    wid = lax.axis_index("s") * NUM_CORES + lax.axis_index("c")
    g0 = wid * NBLK

    base_iota = lax.iota(jnp.int32, 16)
    biota = [base_iota + 16 * j for j in range(8)]

    def hc(g):
        gg = g0 + g
        return lax.shift_right_logical(gg, 7), lax.bitwise_and(gg, 127) * CB

    def fire_idx(g, ib):
        h, c = hc(g)
        pltpu.async_copy(ids_hbm.at[h, pl.ds(c, CB)], idx_v.at[ib], isem.at[ib])

    def wait_idx(g, ib):
        h, c = hc(g)
        pltpu.make_async_copy(
            ids_hbm.at[h, pl.ds(c, CB)], idx_v.at[ib], isem.at[ib]).wait()

    def fire_gather(g, ib, b):
        pltpu.async_copy(tab_hbm.at[idx_v.at[ib]], gb_v.at[b], gsem.at[b])

    def wait_gather(g, ib, b):
        pltpu.make_async_copy(
            tab_hbm.at[idx_v.at[ib]], gb_v.at[b], gsem.at[b]).wait()

    def fire_store(g, b):
        h, c = hc(g)
        pltpu.async_copy(ob_v.at[b], out_hbm.at[h].at[:, pl.ds(c, CB)], ssem.at[b])

    def wait_store(g, b):
        h, c = hc(g)
        pltpu.make_async_copy(
            ob_v.at[b], out_hbm.at[h].at[:, pl.ds(c, CB)], ssem.at[b]).wait()

    def transpose(b):
        def body(d, carry):
            dv = jnp.zeros((16,), jnp.int32) + d
            for j in range(8):
                v = plsc.load_gather(gb_v.at[b], [biota[j], dv])
                ob_v[b, d, pl.ds(16 * j, 16)] = v
            return carry

        lax.fori_loop(0, DIM, body, 0)

    for g in range(NIB):
        fire_idx(g, g)
    for g in range(2):
        wait_idx(g, g)
        fire_gather(g, g, g)

    def step(g, ib, b, ib2, b2, ib8, *, fire_g2=True, fire_i8=True, wait_s4=True):
        if fire_g2:
            wait_idx(g + 2, ib2)
            fire_gather(g + 2, ib2, b2)
        wait_gather(g, ib, b)
        if fire_i8:
            fire_idx(g + NIB, ib8)
        if wait_s4:
            wait_store(g - 1, (b - 1) % NGB)
        transpose(b)
        fire_store(g, b)

    for g in range(8):
        step(g, g % NIB, g % NGB, (g + 2) % NIB, (g + 2) % NGB, g % NIB,
             wait_s4=(g >= 1))

    def outer(t, carry):
        for j in range(8):
            g = 8 + t * 8 + j
            step(g, j % NIB, j % NGB, (j + 2) % NIB, (j + 2) % NGB, j % NIB)
        return carry

    lax.fori_loop(0, (NBLK - 16) // 8, outer, 0)

    for g in range(NBLK - 8, NBLK):
        step(g, g % NIB, g % NGB, (g + 2) % NIB, (g + 2) % NGB, g % NIB,
             fire_g2=(g + 2 < NBLK), fire_i8=False)

    wait_store(NBLK - 1, (NBLK - 1) % NGB)


def kernel(input_ids, table):
    ids_t = input_ids.astype(jnp.int32).T
    tab = jnp.pad(table, ((0, 0), (0, DIM)))
    out_t = _gather_rows(ids_t, tab)
    return out_t.transpose(2, 0, 1)


# speedup vs baseline: 1.1706x; 1.1706x over previous
import functools

import jax
import jax.numpy as jnp
from jax import lax
from jax.experimental import pallas as pl
from jax.experimental.pallas import tpu as pltpu
from jax.experimental.pallas import tpu_sc as plsc

NUM_EMB = 1_000_000
DIM = 64
BATCH = 16384
HIST = 50

NUM_CORES = 2
NUM_SUBCORES = 16
NW = NUM_CORES * NUM_SUBCORES
CB = 128
NBLK_TOTAL = HIST * (BATCH // CB)
NBLK = NBLK_TOTAL // NW

NIB = 8
NGB = 4

_mesh = plsc.VectorSubcoreMesh(core_axis_name="c", subcore_axis_name="s")


@functools.partial(
    pl.kernel,
    mesh=_mesh,
    out_type=jax.ShapeDtypeStruct((HIST, DIM, BATCH), jnp.float32),
    scratch_types=[
        pltpu.VMEM((NIB, CB), jnp.int32),
        pltpu.VMEM((NGB, CB, 2 * DIM), jnp.float32),
        pltpu.VMEM((NGB, DIM, CB), jnp.float32),
        pltpu.SemaphoreType.DMA((NIB,)),
        pltpu.SemaphoreType.DMA((NGB,)),
        pltpu.SemaphoreType.DMA((NGB,)),
    ],
    compiler_params=pltpu.CompilerParams(needs_layout_passes=False),
)
def _gather_rows(ids_hbm, tab_hbm, out_hbm, idx_v, gb_v, ob_v, isem, gsem, ssem):
    wid = lax.axis_index("s") * NUM_CORES + lax.axis_index("c")
    g0 = wid * NBLK

    base_iota = lax.iota(jnp.int32, 16)
    biota = [base_iota + 16 * j for j in range(8)]

    def hc(g):
        gg = g0 + g
        return lax.shift_right_logical(gg, 7), lax.bitwise_and(gg, 127) * CB

    def fire_idx(g, ib):
        h, c = hc(g)
        pltpu.async_copy(ids_hbm.at[h, pl.ds(c, CB)], idx_v.at[ib], isem.at[ib])

    def wait_idx(g, ib):
        h, c = hc(g)
        pltpu.make_async_copy(
            ids_hbm.at[h, pl.ds(c, CB)], idx_v.at[ib], isem.at[ib]).wait()

    def fire_gather(g, ib, b):
        pltpu.async_copy(tab_hbm.at[idx_v.at[ib]], gb_v.at[b], gsem.at[b])

    def wait_gather(g, ib, b):
        pltpu.make_async_copy(
            tab_hbm.at[idx_v.at[ib]], gb_v.at[b], gsem.at[b]).wait()

    def fire_store(g, b):
        h, c = hc(g)
        pltpu.async_copy(ob_v.at[b], out_hbm.at[h].at[:, pl.ds(c, CB)], ssem.at[b])

    def wait_store(g, b):
        h, c = hc(g)
        pltpu.make_async_copy(
            ob_v.at[b], out_hbm.at[h].at[:, pl.ds(c, CB)], ssem.at[b]).wait()

    def transpose(b):
        def body(k, carry):
            dbase = k * 4
            vs = []
            for dd in range(4):
                dv = jnp.zeros((16,), jnp.int32) + (dbase + dd)
                for j in range(8):
                    vs.append(plsc.load_gather(gb_v.at[b], [biota[j], dv]))
            i = 0
            for dd in range(4):
                for j in range(8):
                    ob_v[b, dbase + dd, pl.ds(16 * j, 16)] = vs[i]
                    i += 1
            return carry

        lax.fori_loop(0, DIM // 4, body, 0)

    for g in range(NIB):
        fire_idx(g, g)
    for g in range(2):
        wait_idx(g, g)
        fire_gather(g, g, g)

    def step(g, ib, b, ib2, b2, ib8, *, fire_g2=True, fire_i8=True, wait_s4=True):
        if fire_g2:
            wait_idx(g + 2, ib2)
            fire_gather(g + 2, ib2, b2)
        wait_gather(g, ib, b)
        if fire_i8:
            fire_idx(g + NIB, ib8)
        if wait_s4:
            wait_store(g - 1, (b - 1) % NGB)
        transpose(b)
        fire_store(g, b)

    for g in range(8):
        step(g, g % NIB, g % NGB, (g + 2) % NIB, (g + 2) % NGB, g % NIB,
             wait_s4=(g >= 1))

    def outer(t, carry):
        for j in range(8):
            g = 8 + t * 8 + j
            step(g, j % NIB, j % NGB, (j + 2) % NIB, (j + 2) % NGB, j % NIB)
        return carry

    lax.fori_loop(0, (NBLK - 16) // 8, outer, 0)

    for g in range(NBLK - 8, NBLK):
        step(g, g % NIB, g % NGB, (g + 2) % NIB, (g + 2) % NGB, g % NIB,
             fire_g2=(g + 2 < NBLK), fire_i8=False)

    wait_store(NBLK - 1, (NBLK - 1) % NGB)


def kernel(input_ids, table):
    ids_t = input_ids.astype(jnp.int32).T
    tab = jnp.pad(table, ((0, 0), (0, DIM)))
    out_t = _gather_rows(ids_t, tab)
    return out_t.transpose(2, 0, 1)


# speedup vs baseline: 1.4581x; 1.2455x over previous
import functools

import jax
import jax.numpy as jnp
from jax import lax
from jax.experimental import pallas as pl
from jax.experimental.pallas import tpu as pltpu
from jax.experimental.pallas import tpu_sc as plsc

NUM_EMB = 1_000_000
DIM = 64
BATCH = 16384
HIST = 50
TOTAL = BATCH * HIST

NUM_CORES = 2
NUM_SUBCORES = 16
NW = NUM_CORES * NUM_SUBCORES
PER_W = TOTAL // NW
CH = 128
NSLAB = PER_W // CH

NBUF = 8
LOOK = NBUF // 2

_mesh = plsc.VectorSubcoreMesh(core_axis_name="c", subcore_axis_name="s")


@functools.partial(
    pl.kernel,
    mesh=_mesh,
    out_type=jax.ShapeDtypeStruct((TOTAL, DIM), jnp.float32),
    scratch_types=[
        pltpu.VMEM((NSLAB, CH), jnp.int32),
        pltpu.VMEM((NBUF, CH, DIM), jnp.float32),
        pltpu.SemaphoreType.DMA((NBUF,)),
        pltpu.SemaphoreType.DMA((NBUF,)),
    ],
    compiler_params=pltpu.CompilerParams(use_tc_tiling_on_sc=False),
)
def _gather_rows(ids_hbm, table_hbm, out_hbm, idx_v, rows_v, gsem, ssem):
    wid = lax.axis_index("s") * NUM_CORES + lax.axis_index("c")
    w_base = wid * PER_W

    pltpu.sync_copy(ids_hbm.at[wid], idx_v)

    def fire_gather(s, b):
        pltpu.async_copy(table_hbm.at[idx_v.at[s]], rows_v.at[b], gsem.at[b])

    def drain_gather(s, b):
        pltpu.make_async_copy(
            table_hbm.at[idx_v.at[s]], rows_v.at[b], gsem.at[b]).wait()

    def fire_store(s, b):
        pltpu.async_copy(
            rows_v.at[b], out_hbm.at[pl.ds(w_base + s * CH, CH)], ssem.at[b])

    def wait_store(s, b):
        pltpu.make_async_copy(
            rows_v.at[b], out_hbm.at[pl.ds(w_base + s * CH, CH)], ssem.at[b]).wait()

    for s in range(LOOK):
        fire_gather(s, s)

    for s in range(LOOK):
        drain_gather(s, s)
        fire_store(s, s)
        fire_gather(s + LOOK, (s + LOOK) % NBUF)

    def outer(t, carry):
        for j in range(NBUF):
            s = t * NBUF + j + LOOK
            b = (j + LOOK) % NBUF
            drain_gather(s, b)
            fire_store(s, b)
            wait_store(s - (NBUF - LOOK), j)
            fire_gather(s + LOOK, j)
        return carry

    lax.fori_loop(0, (NSLAB - 2 * LOOK) // NBUF, outer, 0)

    for s in range(NSLAB - LOOK, NSLAB):
        drain_gather(s, s % NBUF)
        fire_store(s, s % NBUF)

    for j in range(NBUF):
        wait_store(NSLAB - NBUF + j, j)


def kernel(input_ids, table):
    ids = input_ids.reshape(NW, NSLAB, CH).astype(jnp.int32)
    out = _gather_rows(ids, table)
    return out.reshape(BATCH, HIST, DIM)
